# baseline (speedup 1.0000x reference)
import functools
import math

import jax
import jax.numpy as jnp
from jax import lax
from jax.experimental import pallas as pl
from jax.experimental.pallas import tpu as pltpu
from jax.experimental.pallas import tpu_sc as plsc

_TINY = 1e-13
_MASK_VAL = -10000.0
_IDX_CHUNK = 96


def _sc_gather(table, bias16, ids2d, n_ids, d):
    info = plsc.get_sparse_core_info()
    nc, ns = info.num_cores, info.num_subcores
    nw = nc * ns
    per_w = n_ids // nw
    chunks = per_w // _IDX_CHUNK
    mesh = plsc.VectorSubcoreMesh(core_axis_name="c", subcore_axis_name="s")

    @functools.partial(
        pl.kernel,
        mesh=mesh,
        out_type=[
            jax.ShapeDtypeStruct((n_ids, d), jnp.float32),
            jax.ShapeDtypeStruct((n_ids,), jnp.float32),
        ],
        scratch_types=[
            pltpu.VMEM((chunks, _IDX_CHUNK), jnp.int32),
            pltpu.VMEM((chunks, _IDX_CHUNK), jnp.int32),
            pltpu.VMEM((per_w, d), jnp.float32),
            pltpu.VMEM((per_w, 16), jnp.float32),
            pltpu.VMEM((per_w,), jnp.float32),
            pltpu.SemaphoreType.DMA,
            pltpu.SemaphoreType.DMA,
        ],
        compiler_params=pltpu.CompilerParams(use_tc_tiling_on_sc=False,
                                             needs_layout_passes=False),
    )
    def gather(table_hbm, bias_hbm, idx_hbm, w_out, b_out, idx_v, hi_v,
               rows_v, b16_v, bflat_v, sem_w, sem_b):
        wid = lax.axis_index("s") * nc + lax.axis_index("c")
        base = wid * per_w
        pltpu.sync_copy(idx_hbm.at[pl.ds(wid * chunks, chunks)], idx_v)
        for r in range(chunks):
            for c in range(_IDX_CHUNK // 16):
                sl = pl.ds(c * 16, 16)
                hi_v[r, sl] = lax.shift_right_logical(idx_v[r, sl], 4)
        copies = []
        for c in range(chunks):
            copies.append(pltpu.async_copy(
                table_hbm.at[idx_v.at[c]],
                rows_v.at[pl.ds(c * _IDX_CHUNK, _IDX_CHUNK)], sem_w))
            copies.append(pltpu.async_copy(
                bias_hbm.at[hi_v.at[c]],
                b16_v.at[pl.ds(c * _IDX_CHUNK, _IDX_CHUNK)], sem_b))
        for cp in copies:
            cp.wait()
        lane_iota = lax.iota(jnp.int32, 16)
        for j in range(per_w // 16):
            off = j * 16
            ids16 = idx_v[off // _IDX_CHUNK, pl.ds(off % _IDX_CHUNK, 16)]
            lane = jnp.bitwise_and(ids16, 15)
            vals = plsc.load_gather(b16_v, [lane_iota + off, lane])
            bflat_v[pl.ds(off, 16)] = vals
        pltpu.sync_copy(rows_v, w_out.at[pl.ds(base, per_w)])
        pltpu.sync_copy(bflat_v, b_out.at[pl.ds(base, per_w)])

    return gather(table, bias16, ids2d)


def _tc_body(nt_ref, emb_ref, tw_ref, sw_ref, tb_ref, sb_ref, tgt_ref,
             sid_ref, out_ref, *, log_nw_p1):
    i = pl.program_id(0)
    nt = nt_ref[0, 0]

    emb = emb_ref[...]
    tw = tw_ref[...]
    sw = sw_ref[...]
    tb = tb_ref[...]
    sb = sb_ref[...]
    tgt = tgt_ref[...]
    sid = sid_ref[...]

    t = tgt.astype(jnp.float32)
    tp = jnp.log((t + 2.0) / (t + 1.0)) * (1.0 / log_nw_p1)
    tec = 1.0 - jnp.exp(nt * jnp.log(1.0 - tp))
    true_logits = (jnp.sum(tw * emb, axis=1, keepdims=True) + tb
                   - jnp.log(tec + _TINY))

    s = sid.astype(jnp.float32)
    sp = jnp.log((s + 2.0) / (s + 1.0)) * (1.0 / log_nw_p1)
    sec = 1.0 - jnp.exp(nt * jnp.log(1.0 - sp))
    col_adj = sb - jnp.log(sec + _TINY)

    logits = lax.dot_general(emb, sw, (((1,), (1,)), ((), ())),
                             preferred_element_type=jnp.float32)
    logits = logits + col_adj
    logits = jnp.where(tgt == sid, _MASK_VAL, logits)

    m = jnp.maximum(jnp.max(logits, axis=1, keepdims=True), true_logits)
    se = (jnp.sum(jnp.exp(logits - m), axis=1, keepdims=True)
          + jnp.exp(true_logits - m))
    lse = m + jnp.log(se)
    part = jnp.sum(lse - true_logits, axis=(0, 1), keepdims=True)

    @pl.when(i == 0)
    def _():
        out_ref[...] = jnp.zeros_like(part)

    out_ref[...] += part


def kernel(embeddings, softmax_w, softmax_b, targets, sampled_ids, num_tries):
    b, d = embeddings.shape
    v = softmax_w.shape[0]
    s = sampled_ids.shape[0]
    n_ids = b + s
    log_nw_p1 = math.log(v + 1)

    all_ids = jnp.concatenate([targets, sampled_ids]).astype(jnp.int32)
    ids2d = all_ids.reshape(n_ids // _IDX_CHUNK, _IDX_CHUNK)
    all_w, all_b = _sc_gather(softmax_w, softmax_b.reshape(v // 16, 16),
                              ids2d, n_ids, d)

    tb = all_b[:b].reshape(b, 1)
    sb = all_b[b:].reshape(1, s)
    tgt2 = targets.reshape(b, 1)
    sid2 = sampled_ids.reshape(1, s)
    nt = jnp.reshape(num_tries, (1, 1)).astype(jnp.float32)

    tile = 256
    grid = (b // tile,)
    out = pl.pallas_call(
        functools.partial(_tc_body, log_nw_p1=log_nw_p1),
        grid=grid,
        in_specs=[
            pl.BlockSpec(memory_space=pltpu.SMEM),
            pl.BlockSpec((tile, d), lambda i: (i, 0)),
            pl.BlockSpec((tile, d), lambda i: (i, 0)),
            pl.BlockSpec((s, d), lambda i: (b // s, 0)),
            pl.BlockSpec((tile, 1), lambda i: (i, 0)),
            pl.BlockSpec((1, s), lambda i: (0, 0)),
            pl.BlockSpec((tile, 1), lambda i: (i, 0)),
            pl.BlockSpec((1, s), lambda i: (0, 0)),
        ],
        out_specs=pl.BlockSpec((1, 1), lambda i: (0, 0)),
        out_shape=jax.ShapeDtypeStruct((1, 1), jnp.float32),
        compiler_params=pltpu.CompilerParams(
            dimension_semantics=("arbitrary",)),
    )(nt, embeddings, all_w, all_w, tb, sb, tgt2, sid2)
    return out[0, 0]
